# Initial kernel scaffold; baseline (speedup 1.0000x reference)
import jax
import jax.numpy as jnp
from jax.experimental import pallas as pl


def kernel(x, table, W_fc, b_fc, W_out, b_out):
    raise NotImplementedError("write your pallas kernel here")


# speedup vs baseline: 10.4109x; 10.4109x over previous
import functools

import jax
import jax.numpy as jnp
from jax import lax
from jax.experimental import pallas as pl
from jax.experimental.pallas import tpu as pltpu
from jax.experimental.pallas import tpu_sc as plsc

B = 4096
L = 50
D = 128
H = 512
O = 128

NC = 2
NS = 16
NW = NC * NS
BPW = B // NW
NBUF = 4
LANES = 16
DG = D // LANES


def _sc_pool_body(x_hbm, table_hbm, out_hbm, idx_v, emb_v, pooled_v, *sems):
    c = lax.axis_index("c")
    s = lax.axis_index("s")
    wid = s * NC + c
    base = wid * BPW

    pltpu.sync_copy(x_hbm.at[pl.ds(base, BPW)], idx_v)

    def gather(b, row):
        return pltpu.make_async_copy(
            table_hbm.at[idx_v.at[row]], emb_v.at[b], sems[b])

    def accum_row(b, row):
        def body_j(j, accs):
            return tuple(
                accs[g] + emb_v[b, j, pl.ds(g * LANES, LANES)]
                for g in range(DG)
            )
        accs = lax.fori_loop(
            0, L, body_j,
            tuple(jnp.zeros((LANES,), jnp.float32) for _ in range(DG)))
        inv = jnp.float32(1.0 / L)
        for g in range(DG):
            pooled_v[row, pl.ds(g * LANES, LANES)] = accs[g] * inv

    for b in range(NBUF):
        gather(b, b).start()

    def outer(i, _):
        g0 = i * NBUF
        for b in range(NBUF):
            gather(b, g0 + b).wait()
            accum_row(b, g0 + b)
            gather(b, g0 + NBUF + b).start()
        return 0

    lax.fori_loop(0, BPW // NBUF - 1, outer, 0)

    g0 = BPW - NBUF
    for b in range(NBUF):
        gather(b, g0 + b).wait()
        accum_row(b, g0 + b)

    pltpu.sync_copy(pooled_v, out_hbm.at[pl.ds(base, BPW)])


@functools.partial(jax.jit, static_argnames=())
def _sc_pool(x, table):
    mesh = plsc.VectorSubcoreMesh(
        core_axis_name="c", subcore_axis_name="s",
        num_cores=NC, num_subcores=NS)
    f = pl.kernel(
        _sc_pool_body,
        out_type=jax.ShapeDtypeStruct((B, D), jnp.float32),
        mesh=mesh,
        scratch_types=(
            [pltpu.VMEM((BPW, L), jnp.int32),
             pltpu.VMEM((NBUF, L, D), jnp.float32),
             pltpu.VMEM((BPW, D), jnp.float32)]
            + [pltpu.SemaphoreType.DMA] * NBUF
        ),
    )
    return f(x, table)


def _mlp_body(p_ref, wfc_ref, bfc_ref, wout_ref, bout_ref, out_ref):
    p = p_ref[...]
    h = lax.dot_general(
        p, wfc_ref[...], (((1,), (1,)), ((), ())),
        preferred_element_type=jnp.float32,
        precision=lax.Precision.HIGHEST,
    ) + bfc_ref[...]
    out_ref[...] = lax.dot_general(
        h, wout_ref[...], (((1,), (1,)), ((), ())),
        preferred_element_type=jnp.float32,
        precision=lax.Precision.HIGHEST,
    ) + bout_ref[...]


def _mlp(pooled, W_fc, b_fc, W_out, b_out):
    BM = 1024
    return pl.pallas_call(
        _mlp_body,
        grid=(B // BM,),
        in_specs=[
            pl.BlockSpec((BM, D), lambda i: (i, 0)),
            pl.BlockSpec((H, D), lambda i: (0, 0)),
            pl.BlockSpec((1, H), lambda i: (0, 0)),
            pl.BlockSpec((O, H), lambda i: (0, 0)),
            pl.BlockSpec((1, O), lambda i: (0, 0)),
        ],
        out_specs=pl.BlockSpec((BM, O), lambda i: (i, 0)),
        out_shape=jax.ShapeDtypeStruct((B, O), jnp.float32),
    )(pooled, W_fc, b_fc.reshape(1, H), W_out, b_out.reshape(1, O))


def kernel(x, table, W_fc, b_fc, W_out, b_out):
    pooled = _sc_pool(x, table)
    return _mlp(pooled, W_fc, b_fc, W_out, b_out)


# speedup vs baseline: 12.7562x; 1.2253x over previous
import functools

import jax
import jax.numpy as jnp
from jax import lax
from jax.experimental import pallas as pl
from jax.experimental.pallas import tpu as pltpu
from jax.experimental.pallas import tpu_sc as plsc

B = 4096
L = 50
D = 128
H = 512
O = 128

NC = 2
NS = 16
NW = NC * NS
BPW = B // NW
NBUF = 4
LANES = 16
DG = D // LANES


def _sc_pool_body(x_hbm, table_hbm, out_hbm, idx_v, emb_v, pooled_v, *sems):
    c = lax.axis_index("c")
    s = lax.axis_index("s")
    wid = s * NC + c
    base = wid * BPW

    pltpu.sync_copy(x_hbm.at[pl.ds(base, BPW)], idx_v)

    def gather(b, row):
        return pltpu.make_async_copy(
            table_hbm.at[idx_v.at[row]], emb_v.at[b], sems[b])

    def accum_row(b, row):
        def body_j(j, accs):
            return tuple(
                accs[g] + emb_v[b, j, pl.ds(g * LANES, LANES)]
                for g in range(DG)
            )
        accs = lax.fori_loop(
            0, L, body_j,
            tuple(jnp.zeros((LANES,), jnp.float32) for _ in range(DG)))
        inv = jnp.float32(1.0 / L)
        for g in range(DG):
            pooled_v[row, pl.ds(g * LANES, LANES)] = accs[g] * inv

    for b in range(NBUF):
        gather(b, b).start()

    def outer(i, _):
        g0 = i * NBUF
        for b in range(NBUF):
            gather(b, g0 + b).wait()
            accum_row(b, g0 + b)
            gather(b, g0 + NBUF + b).start()
        return 0

    lax.fori_loop(0, BPW // NBUF - 1, outer, 0)

    g0 = BPW - NBUF
    for b in range(NBUF):
        gather(b, g0 + b).wait()
        accum_row(b, g0 + b)

    pltpu.sync_copy(pooled_v, out_hbm.at[pl.ds(base, BPW)])


@functools.partial(jax.jit, static_argnames=())
def _sc_pool(x, table):
    mesh = plsc.VectorSubcoreMesh(
        core_axis_name="c", subcore_axis_name="s",
        num_cores=NC, num_subcores=NS)
    f = pl.kernel(
        _sc_pool_body,
        out_type=jax.ShapeDtypeStruct((B, D), jnp.float32),
        mesh=mesh,
        scratch_types=(
            [pltpu.VMEM((BPW, L), jnp.int32),
             pltpu.VMEM((NBUF, L, D), jnp.float32),
             pltpu.VMEM((BPW, D), jnp.float32)]
            + [pltpu.SemaphoreType.DMA] * NBUF
        ),
    )
    return f(x, table)


def _mlp_body(p_ref, wfc_ref, bfc_ref, wout_ref, bout_ref, out_ref):
    wc = lax.dot_general(
        wout_ref[...], wfc_ref[...], (((1,), (0,)), ((), ())),
        preferred_element_type=jnp.float32,
        precision=lax.Precision.HIGHEST,
    )
    bc = lax.dot_general(
        bfc_ref[...], wout_ref[...], (((1,), (1,)), ((), ())),
        preferred_element_type=jnp.float32,
        precision=lax.Precision.HIGHEST,
    ) + bout_ref[...]
    out_ref[...] = lax.dot_general(
        p_ref[...], wc, (((1,), (1,)), ((), ())),
        preferred_element_type=jnp.float32,
        precision=lax.Precision.HIGHEST,
    ) + bc


def _mlp(pooled, W_fc, b_fc, W_out, b_out):
    return pl.pallas_call(
        _mlp_body,
        out_shape=jax.ShapeDtypeStruct((B, O), jnp.float32),
    )(pooled, W_fc, b_fc.reshape(1, H), W_out, b_out.reshape(1, O))


def kernel(x, table, W_fc, b_fc, W_out, b_out):
    pooled = _sc_pool(x, table)
    return _mlp(pooled, W_fc, b_fc, W_out, b_out)


# speedup vs baseline: 14.1988x; 1.1131x over previous
import functools

import jax
import jax.numpy as jnp
from jax import lax
from jax.experimental import pallas as pl
from jax.experimental.pallas import tpu as pltpu
from jax.experimental.pallas import tpu_sc as plsc

B = 4096
L = 50
D = 128
H = 512
O = 128

NC = 2
NS = 16
NW = NC * NS
BPW = B // NW
NBUF = 8
LANES = 16
DG = D // LANES


def _sc_pool_body(x_hbm, table_hbm, out_hbm, idx_v, emb_v, pooled_v, *sems):
    c = lax.axis_index("c")
    s = lax.axis_index("s")
    wid = s * NC + c
    base = wid * BPW

    pltpu.sync_copy(x_hbm.at[pl.ds(base, BPW)], idx_v)

    def gather(b, row):
        return pltpu.make_async_copy(
            table_hbm.at[idx_v.at[row]], emb_v.at[b], sems[b])

    def accum_row(b, row):
        def body_j(j, accs):
            return tuple(
                accs[g] + emb_v[b, j, pl.ds(g * LANES, LANES)]
                for g in range(DG)
            )
        accs = lax.fori_loop(
            0, L, body_j,
            tuple(jnp.zeros((LANES,), jnp.float32) for _ in range(DG)),
            unroll=5)
        inv = jnp.float32(1.0 / L)
        for g in range(DG):
            pooled_v[row, pl.ds(g * LANES, LANES)] = accs[g] * inv

    for b in range(NBUF):
        gather(b, b).start()

    def outer(i, _):
        g0 = i * NBUF
        for b in range(NBUF):
            gather(b, g0 + b).wait()
            accum_row(b, g0 + b)
            gather(b, g0 + NBUF + b).start()
        return 0

    lax.fori_loop(0, BPW // NBUF - 1, outer, 0)

    g0 = BPW - NBUF
    for b in range(NBUF):
        gather(b, g0 + b).wait()
        accum_row(b, g0 + b)

    pltpu.sync_copy(pooled_v, out_hbm.at[pl.ds(base, BPW)])


@functools.partial(jax.jit, static_argnames=())
def _sc_pool(x, table):
    mesh = plsc.VectorSubcoreMesh(
        core_axis_name="c", subcore_axis_name="s",
        num_cores=NC, num_subcores=NS)
    f = pl.kernel(
        _sc_pool_body,
        out_type=jax.ShapeDtypeStruct((B, D), jnp.float32),
        mesh=mesh,
        scratch_types=(
            [pltpu.VMEM((BPW, L), jnp.int32),
             pltpu.VMEM((NBUF, L, D), jnp.float32),
             pltpu.VMEM((BPW, D), jnp.float32)]
            + [pltpu.SemaphoreType.DMA] * NBUF
        ),
    )
    return f(x, table)


def _mlp_body(p_ref, wfc_ref, bfc_ref, wout_ref, bout_ref, out_ref):
    wc = lax.dot_general(
        wout_ref[...], wfc_ref[...], (((1,), (0,)), ((), ())),
        preferred_element_type=jnp.float32,
        precision=lax.Precision.HIGHEST,
    )
    bc = lax.dot_general(
        bfc_ref[...], wout_ref[...], (((1,), (1,)), ((), ())),
        preferred_element_type=jnp.float32,
        precision=lax.Precision.HIGHEST,
    ) + bout_ref[...]
    out_ref[...] = lax.dot_general(
        p_ref[...], wc, (((1,), (1,)), ((), ())),
        preferred_element_type=jnp.float32,
        precision=lax.Precision.HIGHEST,
    ) + bc


def _mlp(pooled, W_fc, b_fc, W_out, b_out):
    return pl.pallas_call(
        _mlp_body,
        out_shape=jax.ShapeDtypeStruct((B, O), jnp.float32),
    )(pooled, W_fc, b_fc.reshape(1, H), W_out, b_out.reshape(1, O))


def kernel(x, table, W_fc, b_fc, W_out, b_out):
    pooled = _sc_pool(x, table)
    return _mlp(pooled, W_fc, b_fc, W_out, b_out)
